# Initial kernel scaffold; baseline (speedup 1.0000x reference)
import jax
import jax.numpy as jnp
from jax.experimental import pallas as pl


def kernel(word_sequences, embedding_table):
    raise NotImplementedError("write your pallas kernel here")


# speedup vs baseline: 4.8065x; 4.8065x over previous
import functools

import jax
import jax.numpy as jnp
from jax import lax
from jax.experimental import pallas as pl
from jax.experimental.pallas import tpu as pltpu
from jax.experimental.pallas import tpu_sc as plsc

_D = 128
_B = 4096 * 200
_NC, _NS = 2, 16
_NW = _NC * _NS
_BPW = _B // _NW
_CH = 128
_M = _BPW // _CH


def _gather_body(idx_hbm, table_hbm, out_hbm, idx_v, rows_v, sem):
    wid = lax.axis_index("s") * _NC + lax.axis_index("c")
    pltpu.sync_copy(idx_hbm.at[wid], idx_v)
    base = pl.multiple_of(wid * _BPW, _CH)

    def body(j, carry):
        pltpu.async_copy(table_hbm.at[idx_v.at[j]], rows_v, sem).wait()
        off = pl.multiple_of(base + j * _CH, _CH)
        pltpu.sync_copy(rows_v, out_hbm.at[pl.ds(off, _CH)])
        return carry

    lax.fori_loop(0, _M, body, 0)


@jax.jit
def _sc_gather(idx3, table):
    k = functools.partial(
        pl.kernel,
        out_type=jax.ShapeDtypeStruct((_B, _D), jnp.float32),
        mesh=plsc.VectorSubcoreMesh(core_axis_name="c", subcore_axis_name="s"),
        scratch_types=[
            pltpu.VMEM((_M, _CH), jnp.int32),
            pltpu.VMEM((_CH, _D), jnp.float32),
            pltpu.SemaphoreType.DMA,
        ],
    )(_gather_body)
    return k(idx3, table)


def kernel(word_sequences, embedding_table):
    idx3 = word_sequences.reshape(_NW, _M, _CH)
    out = _sc_gather(idx3, embedding_table)
    return out.reshape(word_sequences.shape[0], word_sequences.shape[1], _D)


# speedup vs baseline: 5.0434x; 1.0493x over previous
import functools

import jax
import jax.numpy as jnp
from jax import lax
from jax.experimental import pallas as pl
from jax.experimental.pallas import tpu as pltpu
from jax.experimental.pallas import tpu_sc as plsc

_D = 128
_B = 4096 * 200
_NC, _NS = 2, 16
_NW = _NC * _NS
_BPW = _B // _NW
_CH = 128
_M = _BPW // _CH


_NBUF = 4


def _gather_body(idx_hbm, table_hbm, out_hbm, idx_v, rows_v, *sems):
    sem_g, sem_s = sems[:_NBUF], sems[_NBUF:]
    wid = lax.axis_index("s") * _NC + lax.axis_index("c")
    pltpu.sync_copy(idx_hbm.at[wid], idx_v)
    base = pl.multiple_of(wid * _BPW, _CH)

    def out_slice(j):
        return out_hbm.at[pl.ds(pl.multiple_of(base + j * _CH, _CH), _CH)]

    pltpu.async_copy(table_hbm.at[idx_v.at[0]], rows_v.at[0], sem_g[0])
    pltpu.async_copy(table_hbm.at[idx_v.at[1]], rows_v.at[1], sem_g[1])

    def body(g, carry):
        j0 = g * _NBUF
        for b in range(_NBUF):
            j = j0 + b
            bp = (b + 2) % _NBUF
            pltpu.make_async_copy(
                table_hbm.at[idx_v.at[j]], rows_v.at[b], sem_g[b]).wait()
            pltpu.async_copy(rows_v.at[b], out_slice(j), sem_s[b])

            @pl.when(j + 2 < _M)
            def _():
                @pl.when(j >= 2)
                def _():
                    pltpu.make_async_copy(
                        rows_v.at[bp], out_slice(j - 2), sem_s[bp]).wait()
                pltpu.async_copy(
                    table_hbm.at[idx_v.at[j + 2]], rows_v.at[bp], sem_g[bp])
        return carry

    lax.fori_loop(0, _M // _NBUF, body, 0)
    for b in range(_NBUF):
        pltpu.make_async_copy(
            rows_v.at[b], out_slice(_M - _NBUF + b), sem_s[b]).wait()


@jax.jit
def _sc_gather(idx3, table):
    k = functools.partial(
        pl.kernel,
        out_type=jax.ShapeDtypeStruct((_B, _D), jnp.float32),
        mesh=plsc.VectorSubcoreMesh(core_axis_name="c", subcore_axis_name="s"),
        scratch_types=[
            pltpu.VMEM((_M, _CH), jnp.int32),
            pltpu.VMEM((_NBUF, _CH, _D), jnp.float32),
        ] + [pltpu.SemaphoreType.DMA] * (2 * _NBUF),
    )(_gather_body)
    return k(idx3, table)


def kernel(word_sequences, embedding_table):
    idx3 = word_sequences.reshape(_NW, _M, _CH)
    out = _sc_gather(idx3, embedding_table)
    return out.reshape(word_sequences.shape[0], word_sequences.shape[1], _D)


# speedup vs baseline: 15.8822x; 3.1491x over previous
import functools

import jax
import jax.numpy as jnp
from jax import lax
from jax.experimental import pallas as pl
from jax.experimental.pallas import tpu as pltpu
from jax.experimental.pallas import tpu_sc as plsc

_D = 128
_B = 4096 * 200
_NC, _NS = 2, 16
_NW = _NC * _NS
_BPW = _B // _NW
_CH = 128
_M = _BPW // _CH


_NBUF = 4


def _gather_body(idx_hbm, table_hbm, out_hbm, idx_v, rows_v, table_sp, *sems):
    sem_g, sem_s = sems[:_NBUF], sems[_NBUF:]
    sid = lax.axis_index("s")
    wid = sid * _NC + lax.axis_index("c")

    @pl.when(sid == 0)
    def _():
        pltpu.sync_copy(table_hbm, table_sp)
    pltpu.sync_copy(idx_hbm.at[wid], idx_v)
    plsc.subcore_barrier()
    base = pl.multiple_of(wid * _BPW, _CH)

    def out_slice(j):
        return out_hbm.at[pl.ds(pl.multiple_of(base + j * _CH, _CH), _CH)]

    pltpu.async_copy(table_sp.at[idx_v.at[0]], rows_v.at[0], sem_g[0])
    pltpu.async_copy(table_sp.at[idx_v.at[1]], rows_v.at[1], sem_g[1])

    def body(g, carry):
        j0 = g * _NBUF
        for b in range(_NBUF):
            j = j0 + b
            bp = (b + 2) % _NBUF
            pltpu.make_async_copy(
                table_sp.at[idx_v.at[j]], rows_v.at[b], sem_g[b]).wait()
            pltpu.async_copy(rows_v.at[b], out_slice(j), sem_s[b])

            @pl.when(j + 2 < _M)
            def _():
                @pl.when(j >= 2)
                def _():
                    pltpu.make_async_copy(
                        rows_v.at[bp], out_slice(j - 2), sem_s[bp]).wait()
                pltpu.async_copy(
                    table_sp.at[idx_v.at[j + 2]], rows_v.at[bp], sem_g[bp])
        return carry

    lax.fori_loop(0, _M // _NBUF, body, 0)
    for b in range(_NBUF):
        pltpu.make_async_copy(
            rows_v.at[b], out_slice(_M - _NBUF + b), sem_s[b]).wait()


@jax.jit
def _sc_gather(idx3, table):
    k = functools.partial(
        pl.kernel,
        out_type=jax.ShapeDtypeStruct((_B, _D), jnp.float32),
        mesh=plsc.VectorSubcoreMesh(core_axis_name="c", subcore_axis_name="s"),
        scratch_types=[
            pltpu.VMEM((_M, _CH), jnp.int32),
            pltpu.VMEM((_NBUF, _CH, _D), jnp.float32),
            pltpu.VMEM_SHARED((1000, _D), jnp.float32),
        ] + [pltpu.SemaphoreType.DMA] * (2 * _NBUF),
    )(_gather_body)
    return k(idx3, table)


def kernel(word_sequences, embedding_table):
    idx3 = word_sequences.reshape(_NW, _M, _CH)
    out = _sc_gather(idx3, embedding_table)
    return out.reshape(word_sequences.shape[0], word_sequences.shape[1], _D)


# speedup vs baseline: 15.8862x; 1.0003x over previous
import functools

import jax
import jax.numpy as jnp
from jax import lax
from jax.experimental import pallas as pl
from jax.experimental.pallas import tpu as pltpu
from jax.experimental.pallas import tpu_sc as plsc

_D = 128
_B = 4096 * 200
_NC, _NS = 2, 16
_NW = _NC * _NS
_BPW = _B // _NW
_CH = 128
_M = _BPW // _CH


_NBUF = 5
_LEAD = 2


def _gather_body(idx_hbm, table_hbm, out_hbm, idx_v, rows_v, table_sp, *sems):
    sem_g, sem_s = sems[:_NBUF], sems[_NBUF:]
    sid = lax.axis_index("s")
    wid = sid * _NC + lax.axis_index("c")

    @pl.when(sid == 0)
    def _():
        pltpu.sync_copy(table_hbm, table_sp)
    pltpu.sync_copy(idx_hbm.at[wid], idx_v)
    plsc.subcore_barrier()
    base = pl.multiple_of(wid * _BPW, _CH)

    def out_slice(j):
        return out_hbm.at[pl.ds(pl.multiple_of(base + j * _CH, _CH), _CH)]

    for b in range(_LEAD):
        pltpu.async_copy(table_sp.at[idx_v.at[b]], rows_v.at[b], sem_g[b])

    def body(g, carry):
        j0 = g * _NBUF
        for b in range(_NBUF):
            j = j0 + b
            bp = (b + _LEAD) % _NBUF
            pltpu.make_async_copy(
                table_sp.at[idx_v.at[j]], rows_v.at[b], sem_g[b]).wait()
            pltpu.async_copy(rows_v.at[b], out_slice(j), sem_s[b])

            @pl.when(j + _LEAD < _M)
            def _():
                @pl.when(j >= _NBUF - _LEAD)
                def _():
                    pltpu.make_async_copy(
                        rows_v.at[bp], out_slice(j - (_NBUF - _LEAD)),
                        sem_s[bp]).wait()
                pltpu.async_copy(
                    table_sp.at[idx_v.at[j + _LEAD]], rows_v.at[bp], sem_g[bp])
        return carry

    lax.fori_loop(0, _M // _NBUF, body, 0)
    for b in range(_NBUF):
        j = _M - _NBUF + b
        pltpu.make_async_copy(
            rows_v.at[j % _NBUF], out_slice(j), sem_s[j % _NBUF]).wait()


@jax.jit
def _sc_gather(idx3, table):
    k = functools.partial(
        pl.kernel,
        out_type=jax.ShapeDtypeStruct((_B, _D), jnp.float32),
        mesh=plsc.VectorSubcoreMesh(core_axis_name="c", subcore_axis_name="s"),
        scratch_types=[
            pltpu.VMEM((_M, _CH), jnp.int32),
            pltpu.VMEM((_NBUF, _CH, _D), jnp.float32),
            pltpu.VMEM_SHARED((1000, _D), jnp.float32),
        ] + [pltpu.SemaphoreType.DMA] * (2 * _NBUF),
    )(_gather_body)
    return k(idx3, table)


def kernel(word_sequences, embedding_table):
    idx3 = word_sequences.reshape(_NW, _M, _CH)
    out = _sc_gather(idx3, embedding_table)
    return out.reshape(word_sequences.shape[0], word_sequences.shape[1], _D)


# speedup vs baseline: 15.9978x; 1.0070x over previous
import functools

import jax
import jax.numpy as jnp
from jax import lax
from jax.experimental import pallas as pl
from jax.experimental.pallas import tpu as pltpu
from jax.experimental.pallas import tpu_sc as plsc

_D = 128
_B = 4096 * 200
_NC, _NS = 2, 16
_NW = _NC * _NS
_BPW = _B // _NW
_CH = 128
_M = _BPW // _CH


_NBUF = 5
_LEAD = 3


def _gather_body(idx_hbm, table_hbm, out_hbm, idx_v, rows_v, table_sp, *sems):
    sem_g, sem_s = sems[:_NBUF], sems[_NBUF:]
    sid = lax.axis_index("s")
    wid = sid * _NC + lax.axis_index("c")

    @pl.when(sid == 0)
    def _():
        pltpu.sync_copy(table_hbm, table_sp)
    pltpu.sync_copy(idx_hbm.at[wid], idx_v)
    plsc.subcore_barrier()
    base = pl.multiple_of(wid * _BPW, _CH)

    def out_slice(j):
        return out_hbm.at[pl.ds(pl.multiple_of(base + j * _CH, _CH), _CH)]

    for b in range(_LEAD):
        pltpu.async_copy(table_sp.at[idx_v.at[b]], rows_v.at[b], sem_g[b])

    def body(g, carry):
        j0 = g * _NBUF
        for b in range(_NBUF):
            j = j0 + b
            bp = (b + _LEAD) % _NBUF
            pltpu.make_async_copy(
                table_sp.at[idx_v.at[j]], rows_v.at[b], sem_g[b]).wait()
            pltpu.async_copy(rows_v.at[b], out_slice(j), sem_s[b])

            @pl.when(j + _LEAD < _M)
            def _():
                @pl.when(j >= _NBUF - _LEAD)
                def _():
                    pltpu.make_async_copy(
                        rows_v.at[bp], out_slice(j - (_NBUF - _LEAD)),
                        sem_s[bp]).wait()
                pltpu.async_copy(
                    table_sp.at[idx_v.at[j + _LEAD]], rows_v.at[bp], sem_g[bp])
        return carry

    lax.fori_loop(0, _M // _NBUF, body, 0)
    for b in range(_NBUF):
        j = _M - _NBUF + b
        pltpu.make_async_copy(
            rows_v.at[j % _NBUF], out_slice(j), sem_s[j % _NBUF]).wait()


@jax.jit
def _sc_gather(idx3, table):
    k = functools.partial(
        pl.kernel,
        out_type=jax.ShapeDtypeStruct((_B, _D), jnp.float32),
        mesh=plsc.VectorSubcoreMesh(core_axis_name="c", subcore_axis_name="s"),
        scratch_types=[
            pltpu.VMEM((_M, _CH), jnp.int32),
            pltpu.VMEM((_NBUF, _CH, _D), jnp.float32),
            pltpu.VMEM_SHARED((1000, _D), jnp.float32),
        ] + [pltpu.SemaphoreType.DMA] * (2 * _NBUF),
    )(_gather_body)
    return k(idx3, table)


def kernel(word_sequences, embedding_table):
    idx3 = word_sequences.reshape(_NW, _M, _CH)
    out = _sc_gather(idx3, embedding_table)
    return out.reshape(word_sequences.shape[0], word_sequences.shape[1], _D)
